# baseline (speedup 1.0000x reference)
import jax
import jax.numpy as jnp
from jax.experimental import pallas as pl
from jax.experimental.pallas import tpu as pltpu

_VOCAB = 50432
_S = 16
_TOK = 8 * 2048
_T = 256
_CORES = 2
_STEPS = _TOK // (_T * _CORES)
_UNROLL = 64


def _gather_body(ids_ref, table_ref, out_ref, buf, rsem, wsem):
    s = pl.program_id(1)
    block = pl.program_id(0) * _STEPS + s
    base = block * _T
    slot = jax.lax.rem(s, 2)

    @pl.when(s >= 2)
    def _recycle():
        pltpu.make_async_copy(
            buf.at[slot], out_ref.at[pl.ds(0, _T)], wsem.at[slot]
        ).wait()

    def issue(o, carry):
        b = base + o * _UNROLL
        v = o * _UNROLL
        for k in range(_UNROLL):
            idx = ids_ref[b + k]
            pltpu.make_async_copy(
                table_ref.at[idx], buf.at[slot, v + k], rsem
            ).start()
        return carry

    jax.lax.fori_loop(0, _T // _UNROLL, issue, 0)
    pltpu.make_async_copy(
        table_ref.at[pl.ds(0, _T)], buf.at[slot], rsem
    ).wait()

    pltpu.make_async_copy(
        buf.at[slot], out_ref.at[pl.ds(base, _T)], wsem.at[slot]
    ).start(priority=1)

    @pl.when(s == _STEPS - 1)
    def _drain():
        for j in range(2):
            pltpu.make_async_copy(
                buf.at[j], out_ref.at[pl.ds(0, _T)], wsem.at[j]
            ).wait()


def kernel(token_ids, weight_pulse):
    ids = token_ids.reshape(_TOK)
    table = weight_pulse[:_VOCAB].reshape(_VOCAB, _S, 128)
    grid_spec = pltpu.PrefetchScalarGridSpec(
        num_scalar_prefetch=1,
        grid=(_CORES, _STEPS),
        in_specs=[pl.BlockSpec(memory_space=pl.ANY)],
        out_specs=pl.BlockSpec(memory_space=pl.ANY),
        scratch_shapes=[
            pltpu.VMEM((2, _T, _S, 128), jnp.float32),
            pltpu.SemaphoreType.DMA,
            pltpu.SemaphoreType.DMA((2,)),
        ],
    )
    out = pl.pallas_call(
        _gather_body,
        grid_spec=grid_spec,
        out_shape=jax.ShapeDtypeStruct((_TOK, _S, 128), jnp.float32),
        compiler_params=pltpu.CompilerParams(
            dimension_semantics=("parallel", "arbitrary"),
            disable_bounds_checks=True,
        ),
    )(ids, table)
    return out.reshape(8, 2048, 64, 32)


# speedup vs baseline: 1.0026x; 1.0026x over previous
import jax
import jax.numpy as jnp
from jax.experimental import pallas as pl
from jax.experimental.pallas import tpu as pltpu

_ROWS = 65536
_S = 16
_TOK = 8 * 2048
_T = 256
_CORES = 2
_NCHUNK = 2
_CTOK = _TOK // _NCHUNK
_STEPS = _CTOK // (_T * _CORES)
_UNROLL = 64


def _gather_body(ids_ref, table_ref, out_ref, buf, rsem, wsem):
    s = pl.program_id(1)
    block = pl.program_id(0) * _STEPS + s
    base = block * _T
    slot = jax.lax.rem(s, 2)

    @pl.when(s >= 2)
    def _recycle():
        pltpu.make_async_copy(
            buf.at[slot], out_ref.at[pl.ds(0, _T)], wsem.at[slot]
        ).wait()

    def issue(o, carry):
        b = base + o * _UNROLL
        v = o * _UNROLL
        for k in range(_UNROLL):
            idx = ids_ref[b + k]
            pltpu.make_async_copy(
                table_ref.at[idx], buf.at[slot, v + k], rsem
            ).start()
        return carry

    jax.lax.fori_loop(0, _T // _UNROLL, issue, 0)
    pltpu.make_async_copy(
        table_ref.at[pl.ds(0, _T)], buf.at[slot], rsem
    ).wait()

    pltpu.make_async_copy(
        buf.at[slot], out_ref.at[pl.ds(base, _T)], wsem.at[slot]
    ).start(priority=1)

    @pl.when(s == _STEPS - 1)
    def _drain():
        for j in range(2):
            pltpu.make_async_copy(
                buf.at[j], out_ref.at[pl.ds(0, _T)], wsem.at[j]
            ).wait()


def kernel(token_ids, weight_pulse):
    ids = token_ids.reshape(_TOK)
    table = weight_pulse.reshape(_ROWS, _S, 128)
    grid_spec = pltpu.PrefetchScalarGridSpec(
        num_scalar_prefetch=1,
        grid=(_CORES, _STEPS),
        in_specs=[pl.BlockSpec(memory_space=pl.ANY)],
        out_specs=pl.BlockSpec(memory_space=pl.ANY),
        scratch_shapes=[
            pltpu.VMEM((2, _T, _S, 128), jnp.float32),
            pltpu.SemaphoreType.DMA,
            pltpu.SemaphoreType.DMA((2,)),
        ],
    )
    call = pl.pallas_call(
        _gather_body,
        grid_spec=grid_spec,
        out_shape=jax.ShapeDtypeStruct((_CTOK, _S, 128), jnp.float32),
        compiler_params=pltpu.CompilerParams(
            dimension_semantics=("parallel", "arbitrary"),
            disable_bounds_checks=True,
        ),
    )
    pieces = []
    for i in range(_NCHUNK):
        chunk = call(jax.lax.dynamic_slice(ids, (i * _CTOK,), (_CTOK,)), table)
        pieces.append(chunk.reshape(8 // _NCHUNK, 2048, 64, 32))
    return jnp.concatenate(pieces, axis=0)


# speedup vs baseline: 1.2616x; 1.2584x over previous
import jax
import jax.numpy as jnp
from jax.experimental import pallas as pl
from jax.experimental.pallas import tpu as pltpu

_ROWS = 65536
_S = 16
_TOK = 8 * 2048
_T = 256
_CORES = 2
_STEPS = _TOK // (_T * _CORES)
_UNROLL = 64
_NSLOT = 4


def _gather_body(ids_ref, table_ref, out_ref, buf, rsem, wsem):
    s = pl.program_id(1)
    core = pl.program_id(0)
    slot = jax.lax.rem(s, _NSLOT)
    nslot = jax.lax.rem(s + 1, _NSLOT)

    def issue_reads(step, dst_slot):
        base = (core * _STEPS + step) * _T

        def issue(o, carry):
            b = base + o * _UNROLL
            v = o * _UNROLL
            for k in range(_UNROLL):
                idx = ids_ref[b + k]
                pltpu.make_async_copy(
                    table_ref.at[idx], buf.at[dst_slot, v + k], rsem.at[dst_slot]
                ).start()
            return carry

        jax.lax.fori_loop(0, _T // _UNROLL, issue, 0)

    @pl.when(s == 0)
    def _prologue():
        issue_reads(0, slot)

    @pl.when(s >= _NSLOT - 1)
    def _recycle():
        pltpu.make_async_copy(
            buf.at[nslot], out_ref.at[pl.ds(0, _T)], wsem.at[nslot]
        ).wait()

    @pl.when(s + 1 < _STEPS)
    def _ahead():
        issue_reads(s + 1, nslot)

    pltpu.make_async_copy(
        table_ref.at[pl.ds(0, _T)], buf.at[slot], rsem.at[slot]
    ).wait()

    pltpu.make_async_copy(
        buf.at[slot], out_ref.at[pl.ds((core * _STEPS + s) * _T, _T)], wsem.at[slot]
    ).start(priority=1)

    @pl.when(s == _STEPS - 1)
    def _drain():
        for j in range(_NSLOT):
            if j != _STEPS % _NSLOT:
                pltpu.make_async_copy(
                    buf.at[j], out_ref.at[pl.ds(0, _T)], wsem.at[j]
                ).wait()


def kernel(token_ids, weight_pulse):
    ids = token_ids.reshape(_TOK)
    table = weight_pulse.reshape(_ROWS, _S, 128)
    grid_spec = pltpu.PrefetchScalarGridSpec(
        num_scalar_prefetch=1,
        grid=(_CORES, _STEPS),
        in_specs=[pl.BlockSpec(memory_space=pl.ANY)],
        out_specs=pl.BlockSpec(memory_space=pl.ANY),
        scratch_shapes=[
            pltpu.VMEM((_NSLOT, _T, _S, 128), jnp.float32),
            pltpu.SemaphoreType.DMA((_NSLOT,)),
            pltpu.SemaphoreType.DMA((_NSLOT,)),
        ],
    )
    out = pl.pallas_call(
        _gather_body,
        grid_spec=grid_spec,
        out_shape=jax.ShapeDtypeStruct((_TOK, _S, 128), jnp.float32),
        compiler_params=pltpu.CompilerParams(
            dimension_semantics=("parallel", "arbitrary"),
            disable_bounds_checks=True,
        ),
    )(ids, table)
    return out.reshape(8, 2048, 64, 32)


# speedup vs baseline: 1.2727x; 1.0088x over previous
import jax
import jax.numpy as jnp
from jax.experimental import pallas as pl
from jax.experimental.pallas import tpu as pltpu

_ROWS = 65536
_S = 16
_TOK = 8 * 2048
_T = 512
_CORES = 2
_STEPS = _TOK // (_T * _CORES)
_UNROLL = 64
_NSLOT = 4


def _gather_body(ids_ref, table_ref, out_ref, buf, rsem, wsem):
    s = pl.program_id(1)
    core = pl.program_id(0)
    slot = jax.lax.rem(s, _NSLOT)
    nslot = jax.lax.rem(s + 1, _NSLOT)

    def issue_reads(step, dst_slot):
        base = (core * _STEPS + step) * _T

        def issue(o, carry):
            b = base + o * _UNROLL
            v = o * _UNROLL
            for k in range(_UNROLL):
                idx = ids_ref[b + k]
                pltpu.make_async_copy(
                    table_ref.at[idx], buf.at[dst_slot, v + k], rsem.at[dst_slot]
                ).start()
            return carry

        jax.lax.fori_loop(0, _T // _UNROLL, issue, 0)

    @pl.when(s == 0)
    def _prologue():
        issue_reads(0, slot)

    @pl.when(s >= _NSLOT - 1)
    def _recycle():
        pltpu.make_async_copy(
            buf.at[nslot], out_ref.at[pl.ds(0, _T)], wsem.at[nslot]
        ).wait()

    @pl.when(s + 1 < _STEPS)
    def _ahead():
        issue_reads(s + 1, nslot)

    pltpu.make_async_copy(
        table_ref.at[pl.ds(0, _T)], buf.at[slot], rsem.at[slot]
    ).wait()

    pltpu.make_async_copy(
        buf.at[slot], out_ref.at[pl.ds((core * _STEPS + s) * _T, _T)], wsem.at[slot]
    ).start(priority=1)

    @pl.when(s == _STEPS - 1)
    def _drain():
        for j in range(_NSLOT):
            if j != _STEPS % _NSLOT:
                pltpu.make_async_copy(
                    buf.at[j], out_ref.at[pl.ds(0, _T)], wsem.at[j]
                ).wait()


def kernel(token_ids, weight_pulse):
    ids = token_ids.reshape(_TOK)
    table = weight_pulse.reshape(_ROWS, _S, 128)
    grid_spec = pltpu.PrefetchScalarGridSpec(
        num_scalar_prefetch=1,
        grid=(_CORES, _STEPS),
        in_specs=[pl.BlockSpec(memory_space=pl.ANY)],
        out_specs=pl.BlockSpec(memory_space=pl.ANY),
        scratch_shapes=[
            pltpu.VMEM((_NSLOT, _T, _S, 128), jnp.float32),
            pltpu.SemaphoreType.DMA((_NSLOT,)),
            pltpu.SemaphoreType.DMA((_NSLOT,)),
        ],
    )
    out = pl.pallas_call(
        _gather_body,
        grid_spec=grid_spec,
        out_shape=jax.ShapeDtypeStruct((_TOK, _S, 128), jnp.float32),
        compiler_params=pltpu.CompilerParams(
            dimension_semantics=("parallel", "arbitrary"),
            disable_bounds_checks=True,
        ),
    )(ids, table)
    return out.reshape(8, 2048, 64, 32)
